# baseline (speedup 1.0000x reference)
import functools

import jax
import jax.numpy as jnp
from jax import lax
from jax.experimental import pallas as pl
from jax.experimental.pallas import tpu as pltpu
from jax.experimental.pallas import tpu_sc as plsc

B = 64
RX = 1
ANT = 16
T = 14
SC = 1024
P0, P1 = 2, 11

NC, NS, L = 2, 16, 16
NW = NC * NS
NCHUNK = SC // L
INV_DT = 1.0 / float(P1 - P0)
KCH = 4
CB = B // KCH
ROWS = (CB * ANT) // NW


def _in_descs(y0r_hbm, y0i_hbm, y1r_hbm, y1i_hbm, yin_v, sem, b, row):
    bb = row // ANT
    aa = row % ANT
    return (
        (y0r_hbm.at[bb, aa], yin_v.at[b, 0], sem),
        (y0i_hbm.at[bb, aa], yin_v.at[b, 1], sem),
        (y1r_hbm.at[bb, aa], yin_v.at[b, 2], sem),
        (y1i_hbm.at[bb, aa], yin_v.at[b, 3], sem),
    )


def _out_descs(out_v, out_hbm, sem, b, row):
    bb = row // ANT
    aa = row % ANT
    return (
        (out_v.at[b, 0], out_hbm.at[0, bb, 0, aa, 0, 0], sem),
        (out_v.at[b, 1], out_hbm.at[1, bb, 0, aa, 0, 0], sem),
    )


def _sc_body(y0r_hbm, y0i_hbm, y1r_hbm, y1i_hbm, pr_hbm, pi_hbm, out_hbm,
             a_v, yin_v, out_v, s_in0, s_in1, s_out0, s_out1):
    cid = lax.axis_index("c")
    sid = lax.axis_index("s")
    wid = sid * NC + cid
    base = wid * ROWS
    s_in = (s_in0, s_in1)
    s_out = (s_out0, s_out1)

    pltpu.sync_copy(pr_hbm.at[0], yin_v.at[0, 0])
    pltpu.sync_copy(pi_hbm.at[0], yin_v.at[0, 1])
    pltpu.sync_copy(pr_hbm.at[1], yin_v.at[0, 2])
    pltpu.sync_copy(pi_hbm.at[1], yin_v.at[0, 3])

    def factor_body(i, carry):
        s = pl.ds(i * L, L)
        for p in range(2):
            prv = yin_v[0, 2 * p, s]
            piv = yin_v[0, 2 * p + 1, s]
            inv = 1.0 / (prv * prv + piv * piv)
            a_v[2 * p, s] = prv * inv
            a_v[2 * p + 1, s] = piv * inv
        return carry

    lax.fori_loop(0, NCHUNK, factor_body, 0)

    for d in _in_descs(y0r_hbm, y0i_hbm, y1r_hbm, y1i_hbm, yin_v,
                       s_in[0], 0, base):
        pltpu.async_copy(*d)

    @pl.loop(0, ROWS, step=2)
    def row_loop(rr):
        for b in range(2):
            r = rr + b
            row = base + r
            for d in _in_descs(y0r_hbm, y0i_hbm, y1r_hbm, y1i_hbm, yin_v,
                               s_in[b], b, row):
                pltpu.make_async_copy(*d).wait()
            @pl.when(r + 1 < ROWS)
            def _():
                for d in _in_descs(y0r_hbm, y0i_hbm, y1r_hbm, y1i_hbm, yin_v,
                                   s_in[1 - b], 1 - b, row + 1):
                    pltpu.async_copy(*d)
            @pl.when(r >= 2)
            def _():
                for d in _out_descs(out_v, out_hbm, s_out[b], b, row):
                    pltpu.make_async_copy(*d).wait()

            @plsc.parallel_loop(0, NCHUNK, unroll=4)
            def chunk_body(i):
                s = pl.ds(i * L, L)
                y0r = yin_v[b, 0, s]
                y0i = yin_v[b, 1, s]
                y1r = yin_v[b, 2, s]
                y1i = yin_v[b, 3, s]
                a0r = a_v[0, s]
                a0i = a_v[1, s]
                a1r = a_v[2, s]
                a1i = a_v[3, s]
                h0r = y0r * a0r + y0i * a0i
                h0i = y0i * a0r - y0r * a0i
                h1r = y1r * a1r + y1i * a1i
                h1i = y1i * a1r - y1r * a1i
                sr = (h1r - h0r) * INV_DT
                si = (h1i - h0i) * INV_DT
                for t in range(T):
                    w = float(t - P0)
                    out_v[b, 0, t, s] = h0r + w * sr
                    out_v[b, 1, t, s] = h0i + w * si

            for d in _out_descs(out_v, out_hbm, s_out[b], b, row):
                pltpu.async_copy(*d)

    for b in range(2):
        for d in _out_descs(out_v, out_hbm, s_out[b], b, base + ROWS - 2 + b):
            pltpu.make_async_copy(*d).wait()


@jax.jit
def _run(y0r, y0i, y1r, y1i, pr, pi):
    mesh = plsc.VectorSubcoreMesh(core_axis_name="c", subcore_axis_name="s",
                                  num_cores=NC, num_subcores=NS)
    k = functools.partial(
        pl.kernel,
        out_type=jax.ShapeDtypeStruct((2, CB, RX, ANT, 1, 1, T, SC),
                                      jnp.float32),
        mesh=mesh,
        scratch_types=[
            pltpu.VMEM((4, SC), jnp.float32),
            pltpu.VMEM((2, 4, SC), jnp.float32),
            pltpu.VMEM((2, 2, T, SC), jnp.float32),
            pltpu.SemaphoreType.DMA,
            pltpu.SemaphoreType.DMA,
            pltpu.SemaphoreType.DMA,
            pltpu.SemaphoreType.DMA,
        ],
    )(_sc_body)
    outs = []
    for c in range(KCH):
        sl = slice(c * CB, (c + 1) * CB)
        outs.append(k(y0r[sl], y0i[sl], y1r[sl], y1i[sl], pr, pi))
    return jnp.concatenate(outs, axis=1)


def kernel(y_real, y_imag, no, pilots_real, pilots_imag):
    y0r = y_real[:, 0, :, P0]
    y0i = y_imag[:, 0, :, P0]
    y1r = y_real[:, 0, :, P1]
    y1i = y_imag[:, 0, :, P1]
    return _run(y0r, y0i, y1r, y1i, pilots_real, pilots_imag)


# speedup vs baseline: 1.3581x; 1.3581x over previous
import functools

import jax
import jax.numpy as jnp
from jax import lax
from jax.experimental import pallas as pl
from jax.experimental.pallas import tpu as pltpu
from jax.experimental.pallas import tpu_sc as plsc

B = 64
RX = 1
ANT = 16
T = 14
SC = 1024
P0, P1 = 2, 11

NC, NS, L = 2, 16, 16
NW = NC * NS
BA = B * ANT
ROWS = BA // NW
NCHUNK = SC // L
INV_DT = 1.0 / float(P1 - P0)


def _in_descs(y0r_hbm, y0i_hbm, y1r_hbm, y1i_hbm, yin_v, sem, b, row):
    bb = row // ANT
    aa = row % ANT
    return (
        (y0r_hbm.at[bb, aa], yin_v.at[b, 0], sem),
        (y0i_hbm.at[bb, aa], yin_v.at[b, 1], sem),
        (y1r_hbm.at[bb, aa], yin_v.at[b, 2], sem),
        (y1i_hbm.at[bb, aa], yin_v.at[b, 3], sem),
    )


def _sc_body(y0r_hbm, y0i_hbm, y1r_hbm, y1i_hbm, pr_hbm, pi_hbm, hh_hbm,
             a_v, yin_v, hv_v, s_in0, s_in1, s_out0, s_out1):
    cid = lax.axis_index("c")
    sid = lax.axis_index("s")
    wid = sid * NC + cid
    base = wid * ROWS
    s_in = (s_in0, s_in1)
    s_out = (s_out0, s_out1)

    pltpu.sync_copy(pr_hbm.at[0], yin_v.at[0, 0])
    pltpu.sync_copy(pi_hbm.at[0], yin_v.at[0, 1])
    pltpu.sync_copy(pr_hbm.at[1], yin_v.at[0, 2])
    pltpu.sync_copy(pi_hbm.at[1], yin_v.at[0, 3])

    def factor_body(i, carry):
        s = pl.ds(i * L, L)
        for p in range(2):
            prv = yin_v[0, 2 * p, s]
            piv = yin_v[0, 2 * p + 1, s]
            inv = 1.0 / (prv * prv + piv * piv)
            a_v[2 * p, s] = prv * inv
            a_v[2 * p + 1, s] = piv * inv
        return carry

    lax.fori_loop(0, NCHUNK, factor_body, 0)

    for d in _in_descs(y0r_hbm, y0i_hbm, y1r_hbm, y1i_hbm, yin_v,
                       s_in[0], 0, base):
        pltpu.async_copy(*d)

    @pl.loop(0, ROWS, step=2)
    def row_loop(rr):
        for b in range(2):
            r = rr + b
            row = base + r
            for d in _in_descs(y0r_hbm, y0i_hbm, y1r_hbm, y1i_hbm, yin_v,
                               s_in[b], b, row):
                pltpu.make_async_copy(*d).wait()
            @pl.when(r + 1 < ROWS)
            def _():
                for d in _in_descs(y0r_hbm, y0i_hbm, y1r_hbm, y1i_hbm, yin_v,
                                   s_in[1 - b], 1 - b, row + 1):
                    pltpu.async_copy(*d)
            @pl.when(r >= 2)
            def _():
                pltpu.make_async_copy(hv_v.at[b], hh_hbm.at[row], s_out[b]
                                      ).wait()

            @plsc.parallel_loop(0, NCHUNK, unroll=4)
            def chunk_body(i):
                s = pl.ds(i * L, L)
                tr = i // 8
                so = pl.ds((i % 8) * L, L)
                y0r = yin_v[b, 0, s]
                y0i = yin_v[b, 1, s]
                y1r = yin_v[b, 2, s]
                y1i = yin_v[b, 3, s]
                a0r = a_v[0, s]
                a0i = a_v[1, s]
                a1r = a_v[2, s]
                a1i = a_v[3, s]
                h0r = y0r * a0r + y0i * a0i
                h0i = y0i * a0r - y0r * a0i
                h1r = y1r * a1r + y1i * a1i
                h1i = y1i * a1r - y1r * a1i
                hv_v[b, 0, tr, so] = h0r
                hv_v[b, 1, tr, so] = h0i
                hv_v[b, 2, tr, so] = (h1r - h0r) * INV_DT
                hv_v[b, 3, tr, so] = (h1i - h0i) * INV_DT

            pltpu.async_copy(hv_v.at[b], hh_hbm.at[row], s_out[b])

    for b in range(2):
        pltpu.make_async_copy(hv_v.at[b], hh_hbm.at[base + ROWS - 2 + b],
                              s_out[b]).wait()


def _tc_body(hh_ref, out_ref):
    h = hh_ref[...].reshape(ANT, 4, SC)
    tt = (lax.broadcasted_iota(jnp.int32, (1, T, 1), 1).astype(jnp.float32)
          - float(P0))
    h0r = h[:, 0, :][:, None, :]
    h0i = h[:, 1, :][:, None, :]
    sr = h[:, 2, :][:, None, :]
    si = h[:, 3, :][:, None, :]
    out_ref[0, 0, 0, :, 0, 0, :, :] = h0r + tt * sr
    out_ref[1, 0, 0, :, 0, 0, :, :] = h0i + tt * si


@jax.jit
def _run(y0r, y0i, y1r, y1i, pr, pi):
    mesh = plsc.VectorSubcoreMesh(core_axis_name="c", subcore_axis_name="s",
                                  num_cores=NC, num_subcores=NS)
    sc_stage = functools.partial(
        pl.kernel,
        out_type=jax.ShapeDtypeStruct((BA, 4, 8, SC // 8), jnp.float32),
        mesh=mesh,
        scratch_types=[
            pltpu.VMEM((4, SC), jnp.float32),
            pltpu.VMEM((2, 4, SC), jnp.float32),
            pltpu.VMEM((2, 4, 8, SC // 8), jnp.float32),
            pltpu.SemaphoreType.DMA,
            pltpu.SemaphoreType.DMA,
            pltpu.SemaphoreType.DMA,
            pltpu.SemaphoreType.DMA,
        ],
    )(_sc_body)
    hh = sc_stage(y0r, y0i, y1r, y1i, pr, pi)

    tc_stage = pl.pallas_call(
        _tc_body,
        grid=(B,),
        in_specs=[pl.BlockSpec((ANT, 4, 8, SC // 8),
                               lambda b: (b, 0, 0, 0))],
        out_specs=pl.BlockSpec((2, 1, RX, ANT, 1, 1, T, SC),
                               lambda b: (0, b, 0, 0, 0, 0, 0, 0)),
        out_shape=jax.ShapeDtypeStruct((2, B, RX, ANT, 1, 1, T, SC),
                                       jnp.float32),
    )
    return tc_stage(hh)


def kernel(y_real, y_imag, no, pilots_real, pilots_imag):
    y0r = y_real[:, 0, :, P0]
    y0i = y_imag[:, 0, :, P0]
    y1r = y_real[:, 0, :, P1]
    y1i = y_imag[:, 0, :, P1]
    return _run(y0r, y0i, y1r, y1i, pilots_real, pilots_imag)
